# baseline (speedup 1.0000x reference)
import functools

import jax
import jax.numpy as jnp
from jax import lax
from jax.experimental import pallas as pl
from jax.experimental.pallas import tpu as pltpu
from jax.experimental.pallas import tpu_sc as plsc

_EPS = 1e-06
_N_BINS = 64
_ROWS = 8192
_COLS = 512

_S_ROWS = 4096
_T_ROWS = _ROWS - _S_ROWS
_TBLK = 512

_NC = 2
_NS = 16
_NW = _NC * _NS
_LANES = 16

_ROWS_W = _S_ROWS // _NW
_CROWS = 32
_NCHUNK = _ROWS_W // _CROWS
_CHUNK = _CROWS * _COLS

_LO = float(_EPS)
_HI = float(1.0 - _EPS)
_SCALE = float(_N_BINS)


def _sc_body(x_hbm, out_hbm, in0, in1, ot0, ot1, si0, si1, so0, so1):
    wid = lax.axis_index("s") * _NC + lax.axis_index("c")
    base = wid * _ROWS_W
    inbufs = (in0, in1)
    outbufs = (ot0, ot1)
    sin = (si0, si1)
    sout = (so0, so1)

    def start_in(ci, b):
        r0 = base + ci * _CROWS
        return pltpu.async_copy(x_hbm.at[pl.ds(r0, _CROWS)], inbufs[b], sin[b])

    def start_out(ci, b):
        r0 = base + ci * _CROWS
        return pltpu.async_copy(outbufs[b], out_hbm.at[pl.ds(r0, _CROWS)], sout[b])

    def compute(src, dst):
        @plsc.parallel_loop(0, _CROWS, step=1)
        def row_body(r):
            @plsc.parallel_loop(0, _COLS, step=_LANES, unroll=8)
            def col_body(c):
                v = src[r, pl.ds(c, _LANES)]
                v = jnp.minimum(jnp.maximum(v, _LO), _HI) * _SCALE
                dst[r, pl.ds(c, _LANES)] = v.astype(jnp.int32)

    h_in = [start_in(0, 0), start_in(1, 1)]
    h_out = [None, None]
    for ci in range(_NCHUNK):
        b = ci % 2
        h_in[b].wait()
        if ci >= 2:
            h_out[b].wait()
        compute(inbufs[b], outbufs[b])
        h_out[b] = start_out(ci, b)
        if ci + 2 < _NCHUNK:
            h_in[b] = start_in(ci + 2, b)
    h_out[0].wait()
    h_out[1].wait()


_mesh = plsc.VectorSubcoreMesh(core_axis_name="c", subcore_axis_name="s")

_tokenize_sc = functools.partial(
    pl.kernel,
    out_type=jax.ShapeDtypeStruct((_ROWS, _COLS), jnp.int32),
    mesh=_mesh,
    scratch_types=[
        pltpu.VMEM((_CROWS, _COLS), jnp.float32),
        pltpu.VMEM((_CROWS, _COLS), jnp.float32),
        pltpu.VMEM((_CROWS, _COLS), jnp.int32),
        pltpu.VMEM((_CROWS, _COLS), jnp.int32),
        pltpu.SemaphoreType.DMA,
        pltpu.SemaphoreType.DMA,
        pltpu.SemaphoreType.DMA,
        pltpu.SemaphoreType.DMA,
    ],
)(_sc_body)


def _tc_body(x_ref, sc_ref, o_ref):
    del sc_ref
    v = jnp.minimum(jnp.maximum(x_ref[...], _LO), _HI) * _SCALE
    o_ref[...] = v.astype(jnp.int32)


_tokenize_tc = pl.pallas_call(
    _tc_body,
    grid=(_T_ROWS // _TBLK,),
    in_specs=[
        pl.BlockSpec((_TBLK, _COLS), lambda i: (i + _S_ROWS // _TBLK, 0)),
        pl.BlockSpec(memory_space=pl.ANY),
    ],
    out_specs=pl.BlockSpec((_TBLK, _COLS), lambda i: (i + _S_ROWS // _TBLK, 0)),
    out_shape=jax.ShapeDtypeStruct((_ROWS, _COLS), jnp.int32),
    input_output_aliases={1: 0},
)


@jax.jit
def kernel(inputs):
    sc_part = _tokenize_sc(inputs)
    return _tokenize_tc(inputs, sc_part)


# speedup vs baseline: 1.0055x; 1.0055x over previous
import functools

import jax
import jax.numpy as jnp
from jax import lax
from jax.experimental import pallas as pl
from jax.experimental.pallas import tpu as pltpu
from jax.experimental.pallas import tpu_sc as plsc

_EPS = 1e-06
_N_BINS = 64
_ROWS = 8192
_COLS = 512
_PCOLS = _COLS // 4

_S_ROWS = 4096
_T_ROWS = _ROWS - _S_ROWS
_TBLK = 512
_TBLK2 = 512

_NC = 2
_NS = 16
_NW = _NC * _NS
_LANES = 16

_ROWS_W = _S_ROWS // _NW
_CROWS = 32
_NCHUNK = _ROWS_W // _CROWS

_LO = float(_EPS)
_HI = float(1.0 - _EPS)
_SCALE = float(_N_BINS)


def _sc_body(x_hbm, out_hbm, in0, in1, ot0, ot1, si0, si1, so0, so1):
    wid = lax.axis_index("s") * _NC + lax.axis_index("c")
    base = wid * _ROWS_W
    inbufs = (in0, in1)
    outbufs = (ot0, ot1)
    sin = (si0, si1)
    sout = (so0, so1)

    def start_in(ci, b):
        r0 = base + ci * _CROWS
        return pltpu.async_copy(x_hbm.at[pl.ds(r0, _CROWS)], inbufs[b], sin[b])

    def start_out(ci, b):
        r0 = base + ci * _CROWS
        return pltpu.async_copy(outbufs[b], out_hbm.at[pl.ds(r0, _CROWS)], sout[b])

    def compute(src, dst):
        @plsc.parallel_loop(0, _CROWS, step=1)
        def row_body(r):
            @plsc.parallel_loop(0, _PCOLS, step=_LANES, unroll=8)
            def col_body(c):
                def tok(off):
                    v = src[r, pl.ds(c + off, _LANES)]
                    v = jnp.minimum(jnp.maximum(v, _LO), _HI) * _SCALE
                    return v.astype(jnp.int32)

                w = tok(0) | (tok(_PCOLS) << 8) | (tok(2 * _PCOLS) << 16) | (
                    tok(3 * _PCOLS) << 24)
                dst[r, pl.ds(c, _LANES)] = w

    h_in = [start_in(0, 0), start_in(1, 1)]
    h_out = [None, None]
    for ci in range(_NCHUNK):
        b = ci % 2
        h_in[b].wait()
        if ci >= 2:
            h_out[b].wait()
        compute(inbufs[b], outbufs[b])
        h_out[b] = start_out(ci, b)
        if ci + 2 < _NCHUNK:
            h_in[b] = start_in(ci + 2, b)
    h_out[0].wait()
    h_out[1].wait()


_mesh = plsc.VectorSubcoreMesh(core_axis_name="c", subcore_axis_name="s")

_tokenize_sc = functools.partial(
    pl.kernel,
    out_type=jax.ShapeDtypeStruct((_S_ROWS, _PCOLS), jnp.int32),
    mesh=_mesh,
    scratch_types=[
        pltpu.VMEM((_CROWS, _COLS), jnp.float32),
        pltpu.VMEM((_CROWS, _COLS), jnp.float32),
        pltpu.VMEM((_CROWS, _PCOLS), jnp.int32),
        pltpu.VMEM((_CROWS, _PCOLS), jnp.int32),
        pltpu.SemaphoreType.DMA,
        pltpu.SemaphoreType.DMA,
        pltpu.SemaphoreType.DMA,
        pltpu.SemaphoreType.DMA,
    ],
)(_sc_body)


def _tca_body(x_ref, o_ref):
    v = jnp.minimum(jnp.maximum(x_ref[...], _LO), _HI) * _SCALE
    o_ref[...] = v.astype(jnp.int32)


_tokenize_tca = pl.pallas_call(
    _tca_body,
    grid=(_T_ROWS // _TBLK,),
    in_specs=[
        pl.BlockSpec((_TBLK, _COLS), lambda i: (i + _S_ROWS // _TBLK, 0)),
    ],
    out_specs=pl.BlockSpec((_TBLK, _COLS), lambda i: (i + _S_ROWS // _TBLK, 0)),
    out_shape=jax.ShapeDtypeStruct((_ROWS, _COLS), jnp.int32),
)


def _tcb_body(p_ref, tca_ref, o_ref):
    del tca_ref
    p = p_ref[...]
    for j in range(4):
        o_ref[:, _PCOLS * j:_PCOLS * (j + 1)] = (p >> (8 * j)) & (_N_BINS - 1)


_unpack_tcb = pl.pallas_call(
    _tcb_body,
    grid=(_S_ROWS // _TBLK2,),
    in_specs=[
        pl.BlockSpec((_TBLK2, _PCOLS), lambda i: (i, 0)),
        pl.BlockSpec(memory_space=pl.ANY),
    ],
    out_specs=pl.BlockSpec((_TBLK2, _COLS), lambda i: (i, 0)),
    out_shape=jax.ShapeDtypeStruct((_ROWS, _COLS), jnp.int32),
    input_output_aliases={1: 0},
)


@jax.jit
def kernel(inputs):
    packed = _tokenize_sc(inputs)
    tc_part = _tokenize_tca(inputs)
    return _unpack_tcb(packed, tc_part)


# speedup vs baseline: 1.0555x; 1.0498x over previous
import functools

import jax
import jax.numpy as jnp
from jax import lax
from jax.experimental import pallas as pl
from jax.experimental.pallas import tpu as pltpu
from jax.experimental.pallas import tpu_sc as plsc

_EPS = 1e-06
_N_BINS = 64
_ROWS = 8192
_COLS = 512

_NC = 2
_NS = 16
_NW = _NC * _NS
_LANES = 16

_ROWS_W = _ROWS // _NW
_CROWS = 32
_NCHUNK = _ROWS_W // _CROWS

_LO = float(_EPS)
_HI = float(1.0 - _EPS)
_SCALE = float(_N_BINS)


def _sc_body(x_hbm, out_hbm, in0, in1, ot0, ot1, si0, si1, so0, so1):
    wid = lax.axis_index("s") * _NC + lax.axis_index("c")
    base = wid * _ROWS_W
    inbufs = (in0, in1)
    outbufs = (ot0, ot1)
    sin = (si0, si1)
    sout = (so0, so1)

    def start_in(ci, b):
        r0 = base + ci * _CROWS
        return pltpu.async_copy(x_hbm.at[pl.ds(r0, _CROWS)], inbufs[b], sin[b])

    def start_out(ci, b):
        r0 = base + ci * _CROWS
        return pltpu.async_copy(outbufs[b], out_hbm.at[pl.ds(r0, _CROWS)], sout[b])

    def compute(src, dst):
        @plsc.parallel_loop(0, _CROWS, step=1)
        def row_body(r):
            @plsc.parallel_loop(0, _COLS, step=_LANES, unroll=8)
            def col_body(c):
                v = src[r, pl.ds(c, _LANES)]
                v = jnp.minimum(jnp.maximum(v, _LO), _HI) * _SCALE
                dst[r, pl.ds(c, _LANES)] = v.astype(jnp.int32)

    h_in = [start_in(0, 0), start_in(1, 1)]
    h_out = [None, None]
    for ci in range(_NCHUNK):
        b = ci % 2
        h_in[b].wait()
        if ci >= 2:
            h_out[b].wait()
        compute(inbufs[b], outbufs[b])
        h_out[b] = start_out(ci, b)
        if ci + 2 < _NCHUNK:
            h_in[b] = start_in(ci + 2, b)
    h_out[0].wait()
    h_out[1].wait()


_mesh = plsc.VectorSubcoreMesh(core_axis_name="c", subcore_axis_name="s")

_tokenize = functools.partial(
    pl.kernel,
    out_type=jax.ShapeDtypeStruct((_ROWS, _COLS), jnp.int32),
    mesh=_mesh,
    scratch_types=[
        pltpu.VMEM((_CROWS, _COLS), jnp.float32),
        pltpu.VMEM((_CROWS, _COLS), jnp.float32),
        pltpu.VMEM((_CROWS, _COLS), jnp.int32),
        pltpu.VMEM((_CROWS, _COLS), jnp.int32),
        pltpu.SemaphoreType.DMA,
        pltpu.SemaphoreType.DMA,
        pltpu.SemaphoreType.DMA,
        pltpu.SemaphoreType.DMA,
    ],
)(_sc_body)


@jax.jit
def kernel(inputs):
    return _tokenize(inputs)
